# baseline (speedup 1.0000x reference)
import functools

import jax
import jax.numpy as jnp
from jax import lax
from jax.experimental import pallas as pl
from jax.experimental.pallas import tpu as pltpu


def _mm_body(mask_ref, b_ref, w_hbm, a_hbm, o_hbm,
             w_buf, a_buf, o_buf, w_sem, a_sem, o_sem):
    i = pl.program_id(0)
    n = pl.num_programs(0)
    R = a_buf.shape[1]

    def a_copy(blk, slot):
        return pltpu.make_async_copy(
            a_hbm.at[pl.ds(blk * R, R)], a_buf.at[slot], a_sem.at[slot])

    def o_copy(blk, slot):
        return pltpu.make_async_copy(
            o_buf.at[slot], o_hbm.at[pl.ds(blk * R, R)], o_sem.at[slot])

    def w_copy():
        return pltpu.make_async_copy(w_hbm, w_buf, w_sem)

    s = lax.rem(i, 4)
    so = lax.rem(i, 3)

    @pl.when(i == 0)
    def _():
        w_copy().start()
        a_copy(0, 0).start()
        a_copy(1, 1).start()
        a_copy(2, 2).start()

    a_copy(i, s).wait()

    @pl.when(i + 3 < n)
    def _():
        a_copy(i + 3, lax.rem(i + 3, 4)).start()

    @pl.when(i >= 3)
    def _():
        o_copy(i - 3, so).wait()

    @pl.when(i == 0)
    def _():
        w_copy().wait()

    a_bf = a_buf[s].astype(jnp.bfloat16)
    w_bf = w_buf[...].astype(jnp.bfloat16)
    acc = jax.lax.dot_general(
        a_bf, w_bf, (((1,), (1,)), ((), ())),
        preferred_element_type=jnp.float32)
    mask = mask_ref[pl.ds(i * R, R), :]
    o_buf[pl.ds(so, 1)] = ((acc + b_ref[...]) * mask)[None]

    o_copy(i, so).start()

    @pl.when(i == n - 1)
    def _():
        o_copy(i - 2, lax.rem(i - 2, 3)).wait()
        o_copy(i - 1, lax.rem(i - 1, 3)).wait()
        o_copy(i, so).wait()


def kernel(indicators, data, W, b):
    N, d_in = data.shape
    d_out = W.shape[0]
    R = 512
    maskf = (indicators != 0).astype(jnp.float32).reshape(N, 1)
    out = pl.pallas_call(
        _mm_body,
        grid=(N // R,),
        in_specs=[
            pl.BlockSpec((N, 1), lambda i: (0, 0)),
            pl.BlockSpec((1, d_out), lambda i: (0, 0)),
            pl.BlockSpec(memory_space=pl.ANY),
            pl.BlockSpec(memory_space=pl.ANY),
        ],
        out_specs=pl.BlockSpec(memory_space=pl.ANY),
        out_shape=jax.ShapeDtypeStruct((N, d_out), jnp.float32),
        scratch_shapes=[
            pltpu.VMEM((d_out, d_in), jnp.float32),
            pltpu.VMEM((4, R, d_in), jnp.float32),
            pltpu.VMEM((3, R, d_out), jnp.float32),
            pltpu.SemaphoreType.DMA,
            pltpu.SemaphoreType.DMA((4,)),
            pltpu.SemaphoreType.DMA((3,)),
        ],
    )(maskf, b.reshape(1, d_out), W, data)
    return out


# speedup vs baseline: 1.0386x; 1.0386x over previous
import functools

import jax
import jax.numpy as jnp
from jax import lax
from jax.experimental import pallas as pl
from jax.experimental.pallas import tpu as pltpu


def _mm_body(mask_ref, b_ref, w_hbm, a_hbm, o_hbm,
             w_buf, a_buf, o_buf, w_sem, a_sem, o_sem):
    i = pl.program_id(0)
    n = pl.num_programs(0)
    R = a_buf.shape[1]

    def a_copy(blk, slot):
        return pltpu.make_async_copy(
            a_hbm.at[pl.ds(blk * R, R)], a_buf.at[slot], a_sem.at[slot])

    def o_copy(blk, slot):
        return pltpu.make_async_copy(
            o_buf.at[slot], o_hbm.at[pl.ds(blk * R, R)], o_sem.at[slot])

    def w_copy():
        return pltpu.make_async_copy(w_hbm, w_buf, w_sem)

    s = lax.rem(i, 4)
    so = lax.rem(i, 3)

    @pl.when(i == 0)
    def _():
        w_copy().start()
        a_copy(0, 0).start()
        a_copy(1, 1).start()
        a_copy(2, 2).start()

    a_copy(i, s).wait()

    @pl.when(i + 3 < n)
    def _():
        a_copy(i + 3, lax.rem(i + 3, 4)).start()

    @pl.when(i >= 3)
    def _():
        o_copy(i - 3, so).wait()

    @pl.when(i == 0)
    def _():
        w_copy().wait()

    a_bf = a_buf[s].astype(jnp.bfloat16)
    w_bf = w_buf[...].astype(jnp.bfloat16)
    acc = jax.lax.dot_general(
        a_bf, w_bf, (((1,), (1,)), ((), ())),
        preferred_element_type=jnp.float32)
    mask = mask_ref[pl.ds(i * R, R), :]
    o_buf[pl.ds(so, 1)] = ((acc + b_ref[...]) * mask)[None]

    o_copy(i, so).start()

    @pl.when(i == n - 1)
    def _():
        o_copy(i - 2, lax.rem(i - 2, 3)).wait()
        o_copy(i - 1, lax.rem(i - 1, 3)).wait()
        o_copy(i, so).wait()


def kernel(indicators, data, W, b):
    N, d_in = data.shape
    d_out = W.shape[0]
    R = 1024
    maskf = (indicators != 0).astype(jnp.float32).reshape(N, 1)
    out = pl.pallas_call(
        _mm_body,
        grid=(N // R,),
        in_specs=[
            pl.BlockSpec((N, 1), lambda i: (0, 0)),
            pl.BlockSpec((1, d_out), lambda i: (0, 0)),
            pl.BlockSpec(memory_space=pl.ANY),
            pl.BlockSpec(memory_space=pl.ANY),
        ],
        out_specs=pl.BlockSpec(memory_space=pl.ANY),
        out_shape=jax.ShapeDtypeStruct((N, d_out), jnp.float32),
        scratch_shapes=[
            pltpu.VMEM((d_out, d_in), jnp.float32),
            pltpu.VMEM((4, R, d_in), jnp.float32),
            pltpu.VMEM((3, R, d_out), jnp.float32),
            pltpu.SemaphoreType.DMA,
            pltpu.SemaphoreType.DMA((4,)),
            pltpu.SemaphoreType.DMA((3,)),
        ],
    )(maskf, b.reshape(1, d_out), W, data)
    return out


# speedup vs baseline: 1.0914x; 1.0508x over previous
import functools

import jax
import jax.numpy as jnp
from jax import lax
from jax.experimental import pallas as pl
from jax.experimental.pallas import tpu as pltpu


def _mm_body(mask_ref, w_ref, b_ref, a_hbm, o_hbm, a_buf, o_buf, a_sem, o_sem):
    i = pl.program_id(0)
    n = pl.num_programs(0)
    R = a_buf.shape[1]

    def a_copy(blk, slot):
        return pltpu.make_async_copy(
            a_hbm.at[pl.ds(blk * R, R)], a_buf.at[slot], a_sem.at[slot])

    def o_copy(blk, slot):
        return pltpu.make_async_copy(
            o_buf.at[slot], o_hbm.at[pl.ds(blk * R, R)], o_sem.at[slot])

    s = lax.rem(i, 3)
    so = lax.rem(i, 2)

    @pl.when(i == 0)
    def _():
        a_copy(0, 0).start()
        a_copy(1, 1).start()

    @pl.when(i + 2 < n)
    def _():
        a_copy(i + 2, lax.rem(i + 2, 3)).start()

    a_copy(i, s).wait()

    @pl.when(i >= 2)
    def _():
        o_copy(i - 2, so).wait()

    a_bf = a_buf[s].astype(jnp.bfloat16)
    w_bf = w_ref[...].astype(jnp.bfloat16)
    acc = jax.lax.dot_general(
        a_bf, w_bf, (((1,), (1,)), ((), ())),
        preferred_element_type=jnp.float32)
    mask = mask_ref[pl.ds(i * R, R), :]
    o_buf[pl.ds(so, 1)] = ((acc + b_ref[...]) * mask)[None]

    o_copy(i, so).start()

    @pl.when(i == n - 1)
    def _():
        o_copy(i - 1, lax.rem(i - 1, 2)).wait()
        o_copy(i, so).wait()


def kernel(indicators, data, W, b):
    N, d_in = data.shape
    d_out = W.shape[0]
    R = 1024
    maskf = (indicators != 0).astype(jnp.float32).reshape(N, 1)
    out = pl.pallas_call(
        _mm_body,
        grid=(N // R,),
        in_specs=[
            pl.BlockSpec((N, 1), lambda i: (0, 0)),
            pl.BlockSpec((d_out, d_in), lambda i: (0, 0)),
            pl.BlockSpec((1, d_out), lambda i: (0, 0)),
            pl.BlockSpec(memory_space=pl.ANY),
        ],
        out_specs=pl.BlockSpec(memory_space=pl.ANY),
        out_shape=jax.ShapeDtypeStruct((N, d_out), jnp.float32),
        scratch_shapes=[
            pltpu.VMEM((3, R, d_in), jnp.float32),
            pltpu.VMEM((2, R, d_out), jnp.float32),
            pltpu.SemaphoreType.DMA((3,)),
            pltpu.SemaphoreType.DMA((2,)),
        ],
    )(maskf, W, b.reshape(1, d_out), data)
    return out
